# baseline (speedup 1.0000x reference)
import functools

import jax
import jax.numpy as jnp
from jax import lax
from jax.experimental import pallas as pl
from jax.experimental.pallas import tpu as pltpu
from jax.experimental.pallas import tpu_sc as plsc

_EMB = 64
_B = 16384
_NC = 2
_NS = 16
_NW = _NC * _NS
_BPW = _B // _NW
_CHUNK = 128
_NCHUNK = _BPW // _CHUNK


def _sc_gather(uid_ref, iid_ref, uemb_ref, iemb_ref, ubias_ref, ibias_ref,
               ue_out, ie_out, ub_out, ib_out,
               uidx_v, iidx_v, urows_v, irows_v, ub_v, ib_v,
               sem_u, sem_i, sem_ub, sem_ib):
    wid = lax.axis_index("s") * _NC + lax.axis_index("c")
    row_base = wid * _NCHUNK
    base = wid * _BPW

    pltpu.sync_copy(uid_ref.at[pl.ds(row_base, _NCHUNK)], uidx_v)
    pltpu.sync_copy(iid_ref.at[pl.ds(row_base, _NCHUNK)], iidx_v)

    copies = []
    for j in range(_NCHUNK):
        sl = pl.ds(j * _CHUNK, _CHUNK)
        copies.append(pltpu.async_copy(
            uemb_ref.at[uidx_v.at[j]], urows_v.at[sl], sem_u))
        copies.append(pltpu.async_copy(
            iemb_ref.at[iidx_v.at[j]], irows_v.at[sl], sem_i))
        copies.append(pltpu.async_copy(
            ubias_ref.at[uidx_v.at[j]], ub_v.at[sl], sem_ub))
        copies.append(pltpu.async_copy(
            ibias_ref.at[iidx_v.at[j]], ib_v.at[sl], sem_ib))
    for c in copies:
        c.wait()

    bsl = pl.ds(base, _BPW)
    pltpu.sync_copy(urows_v, ue_out.at[bsl])
    pltpu.sync_copy(irows_v, ie_out.at[bsl])
    pltpu.sync_copy(ub_v, ub_out.at[bsl])
    pltpu.sync_copy(ib_v, ib_out.at[bsl])


_sc_gather_call = functools.partial(
    pl.kernel,
    mesh=plsc.VectorSubcoreMesh(core_axis_name="c", subcore_axis_name="s"),
    compiler_params=pltpu.CompilerParams(use_tc_tiling_on_sc=False),
    out_type=[
        jax.ShapeDtypeStruct((_B, _EMB), jnp.float32),
        jax.ShapeDtypeStruct((_B, _EMB), jnp.float32),
        jax.ShapeDtypeStruct((_B,), jnp.float32),
        jax.ShapeDtypeStruct((_B,), jnp.float32),
    ],
    scratch_types=[
        pltpu.VMEM((_NCHUNK, _CHUNK), jnp.int32),
        pltpu.VMEM((_NCHUNK, _CHUNK), jnp.int32),
        pltpu.VMEM((_BPW, _EMB), jnp.float32),
        pltpu.VMEM((_BPW, _EMB), jnp.float32),
        pltpu.VMEM((_BPW,), jnp.float32),
        pltpu.VMEM((_BPW,), jnp.float32),
        pltpu.SemaphoreType.DMA,
        pltpu.SemaphoreType.DMA,
        pltpu.SemaphoreType.DMA,
        pltpu.SemaphoreType.DMA,
    ],
)(_sc_gather)


_BLK = 2048


def _mlp_body(ue_ref, ie_ref, ub_ref, ib_ref, w1a_ref, w1b_ref, b1_ref,
              w2_ref, b2_ref, w3r_ref, gb3_ref, out_ref):
    x = (jnp.dot(ue_ref[...], w1a_ref[...], preferred_element_type=jnp.float32)
         + jnp.dot(ie_ref[...], w1b_ref[...], preferred_element_type=jnp.float32)
         + b1_ref[...])
    h1 = jnp.maximum(x, 0.0)
    h2 = jnp.maximum(
        jnp.dot(h1, w2_ref[...], preferred_element_type=jnp.float32)
        + b2_ref[...], 0.0)
    m = jnp.sum(h2 * w3r_ref[...], axis=1)
    out_ref[...] = m + ub_ref[...] + ib_ref[...] + gb3_ref[...]


def _mlp(ue, ie, ub, ib, w1a, w1b, b1, w2, b2, w3r, gb3):
    grid = (_B // _BLK,)
    return pl.pallas_call(
        _mlp_body,
        grid=grid,
        in_specs=[
            pl.BlockSpec((_BLK, _EMB), lambda i: (i, 0)),
            pl.BlockSpec((_BLK, _EMB), lambda i: (i, 0)),
            pl.BlockSpec((_BLK,), lambda i: (i,)),
            pl.BlockSpec((_BLK,), lambda i: (i,)),
            pl.BlockSpec((_EMB, 128), lambda i: (0, 0)),
            pl.BlockSpec((_EMB, 128), lambda i: (0, 0)),
            pl.BlockSpec((128,), lambda i: (0,)),
            pl.BlockSpec((128, 64), lambda i: (0, 0)),
            pl.BlockSpec((64,), lambda i: (0,)),
            pl.BlockSpec((1, 64), lambda i: (0, 0)),
            pl.BlockSpec((1,), lambda i: (0,)),
        ],
        out_specs=pl.BlockSpec((_BLK,), lambda i: (i,)),
        out_shape=jax.ShapeDtypeStruct((_B,), jnp.float32),
    )(ue, ie, ub, ib, w1a, w1b, b1, w2, b2, w3r, gb3)


def kernel(user_ids, item_ids, user_emb, item_emb, user_bias_t, item_bias_t,
           global_bias, W1, b1, W2, b2, W3, b3):
    uid2d = user_ids.astype(jnp.int32).reshape(_B // _CHUNK, _CHUNK)
    iid2d = item_ids.astype(jnp.int32).reshape(_B // _CHUNK, _CHUNK)
    ubias = user_bias_t.reshape(-1)
    ibias = item_bias_t.reshape(-1)

    ue, ie, ub, ib = _sc_gather_call(
        uid2d, iid2d, user_emb, item_emb, ubias, ibias)

    w1a = W1[:_EMB]
    w1b = W1[_EMB:]
    w3r = W3.reshape(1, -1)
    gb3 = global_bias + b3

    return _mlp(ue, ie, ub, ib, w1a, w1b, b1, W2, b2, w3r, gb3)
